# Initial kernel scaffold; baseline (speedup 1.0000x reference)
import jax
import jax.numpy as jnp
from jax.experimental import pallas as pl


def kernel(x, Wqkv, bqkv, Wlepe, blepe, Wout, bout):
    raise NotImplementedError("write your pallas kernel here")


# speedup vs baseline: 30.4489x; 30.4489x over previous
import jax
import jax.numpy as jnp
import numpy as np
from jax.experimental import pallas as pl
from jax.experimental.pallas import tpu as pltpu

HH, WW, DIM = 28, 28, 512
NH, HD = 8, 64
NWIN, RH = 7, 4
T = HH * WW
R = NWIN * NWIN
RS = RH * RH
TOPK = 4
KK = 8
SCALE = DIM ** -0.5
NEG = -1e30


def _bdot(a, b):
    return jnp.dot(
        a.astype(jnp.bfloat16),
        b.astype(jnp.bfloat16),
        preferred_element_type=jnp.float32,
    )


def _bdot_t(a, b):
    return jax.lax.dot_general(
        a.astype(jnp.bfloat16),
        b.astype(jnp.bfloat16),
        (((1,), (1,)), ((), ())),
        preferred_element_type=jnp.float32,
    )


def _qkv_kernel(x_ref, w_ref, b_ref, o_ref):
    o_ref[0] = _bdot(x_ref[0], w_ref[...]) + b_ref[...]


def _route_kernel(q_ref, k_ref, p_ref, e_ref, o_ref):
    q_r = jnp.dot(
        p_ref[...],
        q_ref[0],
        preferred_element_type=jnp.float32,
        precision=jax.lax.Precision.HIGHEST,
    )
    k_r = jnp.dot(
        p_ref[...],
        k_ref[0],
        preferred_element_type=jnp.float32,
        precision=jax.lax.Precision.HIGHEST,
    )
    a = _bdot_t(q_r, k_r)
    col = jax.lax.broadcasted_iota(jnp.int32, (R, R), 1)
    m = jnp.zeros((R, R), jnp.float32)
    for _ in range(TOPK):
        mx = jnp.max(a, axis=1, keepdims=True)
        cand = jnp.where(a >= mx, col, np.int32(2**30))
        j = jnp.min(cand, axis=1, keepdims=True)
        sel = col == j
        m = jnp.where(sel, 1.0, m)
        a = jnp.where(sel, NEG, a)
    o_ref[0] = jnp.dot(e_ref[...], m, preferred_element_type=jnp.float32)


def _attn_kernel(em_ref, et_ref, q_ref, k_ref, v_ref, o_ref, mask_ref):
    hp = pl.program_id(1)

    @pl.when(hp == 0)
    def _():
        mask_ref[...] = jnp.dot(
            em_ref[0], et_ref[...], preferred_element_type=jnp.float32
        )

    for half in range(2):
        lo, hi = half * HD, (half + 1) * HD
        s = _bdot_t(q_ref[0][:, lo:hi] * SCALE, k_ref[0][:, lo:hi])
        s = jnp.where(mask_ref[...] > 0.5, s, NEG)
        m1 = jnp.max(s, axis=1, keepdims=True)
        cur = s
        for _ in range(KK - 1):
            mx = jnp.max(cur, axis=1, keepdims=True)
            cur = jnp.where(cur >= mx, NEG, cur)
        t8 = jnp.max(cur, axis=1, keepdims=True)
        z = jnp.where(s >= t8, jnp.exp(s - m1), 0.0)
        den = jnp.sum(z, axis=1, keepdims=True)
        o_ref[0, :, lo:hi] = _bdot(z / den, v_ref[0][:, lo:hi])


def _out_kernel(a_ref, v_ref, wl_ref, bl_ref, wo_ref, bo_ref, o_ref):
    v = v_ref[0]
    t = jax.lax.broadcasted_iota(jnp.int32, (T, 1), 0)
    xcol = t % WW
    acc = jnp.zeros((T, DIM), jnp.float32)
    for ky in range(3):
        for kx in range(3):
            s = (ky - 1) * WW + (kx - 1)
            if s > 0:
                sh = jnp.concatenate(
                    [v[s:], jnp.zeros((s, DIM), jnp.float32)], axis=0
                )
            elif s < 0:
                sh = jnp.concatenate(
                    [jnp.zeros((-s, DIM), jnp.float32), v[: T + s]], axis=0
                )
            else:
                sh = v
            row = wl_ref[ky * 3 + kx : ky * 3 + kx + 1, :]
            term = sh * row
            if kx == 0:
                term = jnp.where(xcol == 0, 0.0, term)
            elif kx == 2:
                term = jnp.where(xcol == WW - 1, 0.0, term)
            acc = acc + term
    tmp = a_ref[0] + acc + bl_ref[...]
    o_ref[0] = _bdot(tmp, wo_ref[...]) + bo_ref[...]


def kernel(x, Wqkv, bqkv, Wlepe, blepe, Wout, bout):
    b = x.shape[0]
    xt = x.reshape(b, T, DIM)
    wt = Wqkv.T
    bq = bqkv.reshape(1, 3 * DIM)

    qkv = pl.pallas_call(
        _qkv_kernel,
        grid=(b, 3),
        in_specs=[
            pl.BlockSpec((1, T, DIM), lambda i, j: (i, 0, 0)),
            pl.BlockSpec((DIM, DIM), lambda i, j: (0, j)),
            pl.BlockSpec((1, DIM), lambda i, j: (0, j)),
        ],
        out_specs=pl.BlockSpec((1, T, DIM), lambda i, j: (i, 0, j)),
        out_shape=jax.ShapeDtypeStruct((b, T, 3 * DIM), jnp.float32),
        compiler_params=pltpu.CompilerParams(
            dimension_semantics=("parallel", "arbitrary")
        ),
    )(xt, wt, bq)

    reg = (jnp.arange(T) // WW // RH) * NWIN + (jnp.arange(T) % WW) // RH
    e = (reg[:, None] == jnp.arange(R)[None, :]).astype(jnp.float32)
    p = e.T / RS
    et = e.T

    em = pl.pallas_call(
        _route_kernel,
        grid=(b,),
        in_specs=[
            pl.BlockSpec((1, T, DIM), lambda i: (i, 0, 0)),
            pl.BlockSpec((1, T, DIM), lambda i: (i, 0, 1)),
            pl.BlockSpec((R, T), lambda i: (0, 0)),
            pl.BlockSpec((T, R), lambda i: (0, 0)),
        ],
        out_specs=pl.BlockSpec((1, T, R), lambda i: (i, 0, 0)),
        out_shape=jax.ShapeDtypeStruct((b, T, R), jnp.float32),
        compiler_params=pltpu.CompilerParams(
            dimension_semantics=("arbitrary",)
        ),
    )(qkv, qkv, p, e)

    attn = pl.pallas_call(
        _attn_kernel,
        grid=(b, NH // 2),
        in_specs=[
            pl.BlockSpec((1, T, R), lambda i, h: (i, 0, 0)),
            pl.BlockSpec((R, T), lambda i, h: (0, 0)),
            pl.BlockSpec((1, T, 2 * HD), lambda i, h: (i, 0, h)),
            pl.BlockSpec((1, T, 2 * HD), lambda i, h: (i, 0, h + 4)),
            pl.BlockSpec((1, T, 2 * HD), lambda i, h: (i, 0, h + 8)),
        ],
        out_specs=pl.BlockSpec((1, T, 2 * HD), lambda i, h: (i, 0, h)),
        out_shape=jax.ShapeDtypeStruct((b, T, DIM), jnp.float32),
        scratch_shapes=[pltpu.VMEM((T, T), jnp.float32)],
        compiler_params=pltpu.CompilerParams(
            dimension_semantics=("parallel", "arbitrary")
        ),
    )(em, et, qkv, qkv, qkv)

    wl9 = Wlepe.reshape(DIM, 9).T

    y = pl.pallas_call(
        _out_kernel,
        grid=(b,),
        in_specs=[
            pl.BlockSpec((1, T, DIM), lambda i: (i, 0, 0)),
            pl.BlockSpec((1, T, DIM), lambda i: (i, 0, 2)),
            pl.BlockSpec((9, DIM), lambda i: (0, 0)),
            pl.BlockSpec((1, DIM), lambda i: (0, 0)),
            pl.BlockSpec((DIM, DIM), lambda i: (0, 0)),
            pl.BlockSpec((1, DIM), lambda i: (0, 0)),
        ],
        out_specs=pl.BlockSpec((1, T, DIM), lambda i: (i, 0, 0)),
        out_shape=jax.ShapeDtypeStruct((b, T, DIM), jnp.float32),
        compiler_params=pltpu.CompilerParams(
            dimension_semantics=("parallel",)
        ),
    )(attn, qkv, wl9, blepe.reshape(1, DIM), Wout.T, bout.reshape(1, DIM))
    return y.reshape(b, HH, WW, DIM)
